# baseline (device time: 20071 ns/iter reference)
import jax
import jax.numpy as jnp
from jax import lax
from jax.experimental import pallas as pl
from jax.experimental.pallas import tpu as pltpu

N_DEV = 8
N_TOK = 512
D_IN = 256
D_OUT = 512
E_PER = 2
C = N_TOK // N_DEV
CAP = 32


def kernel(x, router_W, route_idx, expert_W):
    del router_W

    def body(x_ref, idx_ref, w_ref, out_ref,
             x2buf, pbuf, rs_buf, agbuf, ag_buf,
             rs_send, rs_recv, ag_send, ag_recv):
        my = lax.axis_index("i").astype(jnp.int32)

        barrier_sem = pltpu.get_barrier_semaphore()
        for j in range(1, N_DEV):
            pl.semaphore_signal(
                barrier_sem, inc=1,
                device_id=((my + j) % N_DEV,),
                device_id_type=pl.DeviceIdType.MESH,
            )

        e0 = 2 * my
        idx = idx_ref[:, :]
        x_val = x_ref[:, :]
        x0 = jnp.where(idx == e0, x_val, 0.0)
        x1 = jnp.where(idx == e0 + 1, x_val, 0.0)
        x2buf[:, :] = jnp.concatenate([x0, x1], axis=1).astype(jnp.bfloat16)
        wcat = w_ref[:, :, :].reshape(E_PER * D_IN, D_OUT).astype(jnp.bfloat16)

        r0 = lax.broadcasted_iota(jnp.int32, (C, C), 0)
        r1 = lax.broadcasted_iota(jnp.int32, (C, C), 1)
        L64 = (r1 <= r0).astype(jnp.bfloat16)
        kvals = lax.broadcasted_iota(jnp.int32, (1, CAP), 1) + 1

        def onehot(block_off, ea):
            idxb = idx_ref[pl.ds(block_off, C), :]
            maskb = ((idxb == ea) | (idxb == ea + 1))
            maskf = maskb.astype(jnp.bfloat16)
            rank = jnp.dot(L64, maskf, preferred_element_type=jnp.float32)
            rankm = jnp.where(maskb, rank.astype(jnp.int32), 0)
            return (rankm == kvals).astype(jnp.bfloat16)

        def compact_chunk(t):
            g = onehot(C * t, e0)
            x2blk = x2buf[pl.ds(C * t, C), :]
            xg = lax.dot_general(
                g, x2blk, (((0,), (0,)), ((), ())),
                preferred_element_type=jnp.float32,
            ).astype(jnp.bfloat16)
            return jnp.dot(xg, wcat, preferred_element_type=jnp.float32)

        pbuf[0, :, :] = compact_chunk((my + 1) % N_DEV).astype(jnp.bfloat16)

        pl.semaphore_wait(barrier_sem, N_DEV - 1)

        rs_list = []
        for j in range(N_DEV - 1):
            t = (my + j + 1) % N_DEV
            rdma = pltpu.make_async_remote_copy(
                src_ref=pbuf.at[j],
                dst_ref=rs_buf.at[N_DEV - 2 - j],
                send_sem=rs_send.at[j],
                recv_sem=rs_recv.at[N_DEV - 2 - j],
                device_id=(t,),
                device_id_type=pl.DeviceIdType.MESH,
            )
            rdma.start()
            rs_list.append(rdma)
            if j + 1 < N_DEV - 1:
                tn = (my + j + 2) % N_DEV
                pbuf[j + 1, :, :] = compact_chunk(tn).astype(jnp.bfloat16)

        my_off = C * my
        acc = jnp.dot(
            x2buf[pl.ds(my_off, C), :], wcat,
            preferred_element_type=jnp.float32,
        )

        for j, r in enumerate(rs_list):
            r.wait_recv()
            slot = N_DEV - 2 - j
            m = (my + slot + 1) % N_DEV
            s = onehot(my_off, 2 * m)
            acc = acc + jnp.dot(
                s, rs_buf[slot, :, :], preferred_element_type=jnp.float32
            )
        out_ref[pl.ds(my_off, C), :] = acc
        agbuf[:, :] = acc.astype(jnp.bfloat16)

        ag_list = []
        for j in range(N_DEV - 1):
            t = (my + j + 1) % N_DEV
            rdma = pltpu.make_async_remote_copy(
                src_ref=agbuf,
                dst_ref=ag_buf.at[N_DEV - 2 - j],
                send_sem=ag_send.at[j],
                recv_sem=ag_recv.at[N_DEV - 2 - j],
                device_id=(t,),
                device_id_type=pl.DeviceIdType.MESH,
            )
            rdma.start()
            ag_list.append(rdma)
        for j in range(N_DEV - 2, -1, -1):
            r = ag_list[N_DEV - 2 - j]
            r.wait_recv()
            m = (my + j + 1) % N_DEV
            out_ref[pl.ds(C * m, C), :] = ag_buf[j, :, :].astype(jnp.float32)
        for r in rs_list:
            r.wait_send()
        for r in ag_list:
            r.wait_send()

    return pl.pallas_call(
        body,
        out_shape=jax.ShapeDtypeStruct((N_TOK, D_OUT), jnp.float32),
        in_specs=[
            pl.BlockSpec(memory_space=pltpu.VMEM),
            pl.BlockSpec(memory_space=pltpu.VMEM),
            pl.BlockSpec(memory_space=pltpu.VMEM),
        ],
        out_specs=pl.BlockSpec(memory_space=pltpu.VMEM),
        scratch_shapes=[
            pltpu.VMEM((N_TOK, 2 * D_IN), jnp.bfloat16),
            pltpu.VMEM((N_DEV - 1, CAP, D_OUT), jnp.bfloat16),
            pltpu.VMEM((N_DEV - 1, CAP, D_OUT), jnp.bfloat16),
            pltpu.VMEM((C, D_OUT), jnp.bfloat16),
            pltpu.VMEM((N_DEV - 1, C, D_OUT), jnp.bfloat16),
            pltpu.SemaphoreType.DMA((N_DEV - 1,)),
            pltpu.SemaphoreType.DMA((N_DEV - 1,)),
            pltpu.SemaphoreType.DMA((N_DEV - 1,)),
            pltpu.SemaphoreType.DMA((N_DEV - 1,)),
        ],
        compiler_params=pltpu.CompilerParams(collective_id=0),
    )(x, route_idx, expert_W)
